# baseline (speedup 1.0000x reference)
import functools

import jax
import jax.numpy as jnp
from jax import lax
from jax.experimental import pallas as pl
from jax.experimental.pallas import tpu as pltpu
from jax.experimental.pallas import tpu_sc as plsc

NC = 2
NS = 16
NW = NC * NS
D = 64
CH = 512
NBUF = 2


def _table_transpose(table_t):
    V = table_t.shape[1]
    BN = 2048
    grid = (V + BN - 1) // BN

    def body(x_ref, o_ref):
        o_ref[...] = x_ref[...].T

    return pl.pallas_call(
        body,
        grid=(grid,),
        in_specs=[pl.BlockSpec((D, BN), lambda i: (0, i))],
        out_specs=pl.BlockSpec((BN, D), lambda i: (i, 0)),
        out_shape=jax.ShapeDtypeStruct((V, D), jnp.float32),
    )(table_t)


def _out_transpose(rows):
    S, T, _ = rows.shape
    BT = 512

    def body(x_ref, o_ref):
        o_ref[...] = jnp.swapaxes(x_ref[...], 1, 2)

    return pl.pallas_call(
        body,
        grid=(S, T // BT),
        in_specs=[pl.BlockSpec((1, BT, D), lambda j, t: (j, t, 0))],
        out_specs=pl.BlockSpec((1, D, BT), lambda j, t: (j, 0, t)),
        out_shape=jax.ShapeDtypeStruct((S, D, T), jnp.float32),
    )(rows)


def _gather_kernel(B):
    bw = B // NW
    n_ch = bw // CH
    assert bw % CH == 0 and n_ch % NBUF == 0
    mesh = plsc.VectorSubcoreMesh(
        core_axis_name="c", subcore_axis_name="s",
        num_cores=NC, num_subcores=NS)

    @functools.partial(
        pl.kernel,
        out_type=jax.ShapeDtypeStruct((B, D), jnp.float32),
        mesh=mesh,
        compiler_params=pltpu.CompilerParams(use_tc_tiling_on_sc=False),
        scratch_types=[
            pltpu.VMEM((CH,), jnp.int32),
            pltpu.VMEM((CH,), jnp.int32),
            pltpu.VMEM((CH, D), jnp.float32),
            pltpu.VMEM((CH, D), jnp.float32),
            pltpu.SemaphoreType.DMA,
            pltpu.SemaphoreType.DMA,
        ],
    )
    def body(idx_hbm, table_hbm, out_hbm, idx0, idx1, rows0, rows1,
             sem0, sem1):
        idxs = (idx0, idx1)
        rows = (rows0, rows1)
        sems = (sem0, sem1)
        wid = lax.axis_index("s") * NC + lax.axis_index("c")
        base = wid * bw

        for b in range(NBUF):
            pltpu.sync_copy(idx_hbm.at[pl.ds(base + b * CH, CH)], idxs[b])
            pltpu.async_copy(table_hbm.at[idxs[b]], rows[b], sems[b])

        def pair(g, _):
            for b in range(NBUF):
                i = g * NBUF + b
                pltpu.make_async_copy(
                    table_hbm.at[idxs[b]], rows[b], sems[b]).wait()
                pltpu.sync_copy(
                    rows[b], out_hbm.at[pl.ds(base + i * CH, CH)])

                @pl.when(i + NBUF < n_ch)
                def _():
                    pltpu.sync_copy(
                        idx_hbm.at[pl.ds(base + (i + NBUF) * CH, CH)],
                        idxs[b])
                    pltpu.async_copy(
                        table_hbm.at[idxs[b]], rows[b], sems[b])
            return _

        lax.fori_loop(0, n_ch // NBUF, pair, None)

    return body


def kernel(token_ids, embedding):
    T, S = token_ids.shape
    B = T * S
    idx = jnp.reshape(token_ids.T, (B,)).astype(jnp.int32)
    table_rm = _table_transpose(embedding.T)
    out_rm = _gather_kernel(B)(idx, table_rm)
    out_p = _out_transpose(jnp.reshape(out_rm, (S, T, D)))
    return jnp.transpose(out_p, (2, 0, 1))


# speedup vs baseline: 2.6279x; 2.6279x over previous
import functools

import jax
import jax.numpy as jnp
from jax import lax
from jax.experimental import pallas as pl
from jax.experimental.pallas import tpu as pltpu
from jax.experimental.pallas import tpu_sc as plsc

NC = 2
NS = 16
NW = NC * NS
D = 64
TB = 128
CPB = 2
CT = CPB * TB
PAD = 3
NBUF = 2


def _gather_kernel(S, T):
    n_tb = T // TB
    n_tw = n_tb // NW
    n_ch = S * n_tw // CPB
    assert n_tb % NW == 0 and n_tw % CPB == 0 and n_ch % NBUF == 0
    mesh = plsc.VectorSubcoreMesh(
        core_axis_name="c", subcore_axis_name="s",
        num_cores=NC, num_subcores=NS)

    @functools.partial(
        pl.kernel,
        out_type=jax.ShapeDtypeStruct((S, D // 8, n_tb, 8, TB), jnp.float32),
        mesh=mesh,
        compiler_params=pltpu.CompilerParams(
            use_tc_tiling_on_sc=False, needs_layout_passes=False),
        scratch_types=(
            [pltpu.VMEM((S, n_tw * TB), jnp.int32)]
            + [pltpu.VMEM((CT, D), jnp.float32) for _ in range(NBUF)]
            + [pltpu.VMEM((D, CT + PAD), jnp.float32) for _ in range(NBUF)]
            + [pltpu.SemaphoreType.DMA for _ in range(2 * NBUF)]
        ),
    )
    def body(idx_hbm, table_hbm, out_hbm, idx_all, *bufs):
        rows = bufs[0:NBUF]
        tbufs = bufs[NBUF:2 * NBUF]
        gsems = bufs[2 * NBUF:3 * NBUF]
        wsems = bufs[3 * NBUF:4 * NBUF]
        wid = lax.axis_index("s") * NC + lax.axis_index("c")
        t0 = wid * n_tw

        iot = lax.iota(jnp.int32, 16)
        fidx = [fb * 16 + iot for fb in range(D // 16)]

        pltpu.sync_copy(
            idx_hbm.at[:, pl.ds(t0 * TB, n_tw * TB)], idx_all)

        def chunk_idx(i):
            return i % S, i // S

        for b in range(NBUF):
            j0, tp0 = chunk_idx(b)
            pltpu.async_copy(
                table_hbm.at[idx_all.at[j0, pl.ds(tp0 * CT, CT)]],
                rows[b], gsems[b])

        def step(grp, _):
            for b in range(NBUF):
                i = grp * NBUF + b
                j, tp = chunk_idx(i)
                pltpu.make_async_copy(
                    table_hbm.at[idx_all.at[j, pl.ds(tp * CT, CT)]],
                    rows[b], gsems[b]).wait()

                @pl.when(i >= NBUF)
                def _():
                    for g in range(D // 8):
                        for h in range(CPB):
                            pltpu.make_async_copy(
                                tbufs[b].at[pl.ds(g * 8, 8),
                                            pl.ds(h * TB, TB)],
                                out_hbm.at[j, g, t0 + tp * CPB + h],
                                wsems[b]).wait()

                def tloop(c, _c):
                    csp = iot * 0 + c
                    for fb in range(D // 16):
                        v = rows[b][c, pl.ds(fb * 16, 16)]
                        plsc.store_scatter(tbufs[b], [fidx[fb], csp], v)
                    return _c

                lax.fori_loop(0, CT, tloop, None, unroll=16)

                for g in range(D // 8):
                    for h in range(CPB):
                        pltpu.async_copy(
                            tbufs[b].at[pl.ds(g * 8, 8), pl.ds(h * TB, TB)],
                            out_hbm.at[j, g, t0 + tp * CPB + h], wsems[b])

                @pl.when(i + NBUF < n_ch)
                def _():
                    jn, tpn = chunk_idx(i + NBUF)
                    pltpu.async_copy(
                        table_hbm.at[idx_all.at[jn, pl.ds(tpn * CT, CT)]],
                        rows[b], gsems[b])
            return _

        lax.fori_loop(0, n_ch // NBUF, step, None)

        for b in range(NBUF):
            j, tp = chunk_idx(n_ch - NBUF + b)
            for g in range(D // 8):
                for h in range(CPB):
                    pltpu.make_async_copy(
                        tbufs[b].at[pl.ds(g * 8, 8), pl.ds(h * TB, TB)],
                        out_hbm.at[j, g, t0 + tp * CPB + h], wsems[b]).wait()

    return body


def kernel(token_ids, embedding):
    T, S = token_ids.shape
    idx = token_ids.T.astype(jnp.int32)
    out5 = _gather_kernel(S, T)(idx, embedding)
    return jnp.reshape(jnp.transpose(out5, (2, 4, 0, 1, 3)), (T, S, D))


# speedup vs baseline: 2.6327x; 1.0018x over previous
import functools

import jax
import jax.numpy as jnp
from jax import lax
from jax.experimental import pallas as pl
from jax.experimental.pallas import tpu as pltpu
from jax.experimental.pallas import tpu_sc as plsc

NC = 2
NS = 16
NW = NC * NS
D = 64
TB = 128
NBUF = 4


def _gather_kernel(S, T):
    n_tb = T // TB
    n_tw = n_tb // NW
    n_ch = S * n_tw
    assert n_tb % NW == 0 and n_ch % NBUF == 0
    mesh = plsc.VectorSubcoreMesh(
        core_axis_name="c", subcore_axis_name="s",
        num_cores=NC, num_subcores=NS)

    @functools.partial(
        pl.kernel,
        out_type=jax.ShapeDtypeStruct((S, D // 8, n_tb, 8, TB), jnp.float32),
        mesh=mesh,
        compiler_params=pltpu.CompilerParams(
            use_tc_tiling_on_sc=False, needs_layout_passes=False),
        scratch_types=(
            [pltpu.VMEM((S, n_tw * TB), jnp.int32)]
            + [pltpu.VMEM((TB, D), jnp.float32) for _ in range(NBUF)]
            + [pltpu.VMEM((D // 8, 8, TB + 1), jnp.float32)
               for _ in range(NBUF)]
            + [pltpu.SemaphoreType.DMA for _ in range(2 * NBUF)]
        ),
    )
    def body(idx_hbm, table_hbm, out_hbm, idx_all, *bufs):
        rows = bufs[0:NBUF]
        tbufs = bufs[NBUF:2 * NBUF]
        gsems = bufs[2 * NBUF:3 * NBUF]
        wsems = bufs[3 * NBUF:4 * NBUF]
        wid = lax.axis_index("s") * NC + lax.axis_index("c")
        t0 = wid * n_tw

        iot = lax.iota(jnp.int32, 16)
        gidx = [(fb * 16 + iot) // 8 for fb in range(D // 16)]
        ridx = [(fb * 16 + iot) % 8 for fb in range(D // 16)]

        pltpu.sync_copy(
            idx_hbm.at[:, pl.ds(t0 * TB, n_tw * TB)], idx_all)

        def chunk_idx(i):
            return i % S, i // S

        for b in range(NBUF):
            j0, tl0 = chunk_idx(b)
            pltpu.async_copy(
                table_hbm.at[idx_all.at[j0, pl.ds(tl0 * TB, TB)]],
                rows[b], gsems[b])

        def step(grp, _):
            for b in range(NBUF):
                i = grp * NBUF + b
                j, tl = chunk_idx(i)
                t = t0 + tl
                pltpu.make_async_copy(
                    table_hbm.at[idx_all.at[j, pl.ds(tl * TB, TB)]],
                    rows[b], gsems[b]).wait()

                @pl.when(i >= NBUF)
                def _():
                    pltpu.make_async_copy(
                        tbufs[b].at[:, :, pl.ds(0, TB)],
                        out_hbm.at[j, :, t], wsems[b]).wait()

                def tloop(c, _c):
                    csp = iot * 0 + c
                    for fb in range(D // 16):
                        v = rows[b][c, pl.ds(fb * 16, 16)]
                        plsc.store_scatter(
                            tbufs[b], [gidx[fb], ridx[fb], csp], v)
                    return _c

                lax.fori_loop(0, TB, tloop, None, unroll=16)

                pltpu.async_copy(
                    tbufs[b].at[:, :, pl.ds(0, TB)],
                    out_hbm.at[j, :, t], wsems[b])

                @pl.when(i + NBUF < n_ch)
                def _():
                    jn, tln = chunk_idx(i + NBUF)
                    pltpu.async_copy(
                        table_hbm.at[idx_all.at[jn, pl.ds(tln * TB, TB)]],
                        rows[b], gsems[b])
            return _

        lax.fori_loop(0, n_ch // NBUF, step, None)

        for b in range(NBUF):
            j, tl = chunk_idx(n_ch - NBUF + b)
            t = t0 + tl
            pltpu.make_async_copy(
                tbufs[b].at[:, :, pl.ds(0, TB)],
                out_hbm.at[j, :, t], wsems[b]).wait()

    return body


def kernel(token_ids, embedding):
    T, S = token_ids.shape
    idx = token_ids.T.astype(jnp.int32)
    out5 = _gather_kernel(S, T)(idx, embedding)
    return jnp.reshape(jnp.transpose(out5, (2, 4, 0, 1, 3)), (T, S, D))


# speedup vs baseline: 2.6662x; 1.0127x over previous
import functools

import jax
import jax.numpy as jnp
from jax import lax
from jax.experimental import pallas as pl
from jax.experimental.pallas import tpu as pltpu
from jax.experimental.pallas import tpu_sc as plsc

NC = 2
NS = 16
NW = NC * NS
D = 64
TB = 128
NSUB = 4
CT = NSUB * TB
NBUF = 2


def _gather_kernel(S, T):
    n_tb = T // TB
    assert n_tb == NSUB * NW
    mesh = plsc.VectorSubcoreMesh(
        core_axis_name="c", subcore_axis_name="s",
        num_cores=NC, num_subcores=NS)

    @functools.partial(
        pl.kernel,
        out_type=jax.ShapeDtypeStruct((S, D // 8, n_tb, 8, TB), jnp.float32),
        mesh=mesh,
        compiler_params=pltpu.CompilerParams(
            use_tc_tiling_on_sc=False, needs_layout_passes=False),
        scratch_types=(
            [pltpu.VMEM((CT,), jnp.int32) for _ in range(NBUF)]
            + [pltpu.VMEM((CT, D), jnp.float32) for _ in range(NBUF)]
            + [pltpu.VMEM((D // 8, 8, TB + 1), jnp.float32)
               for _ in range(NSUB)]
            + [pltpu.SemaphoreType.DMA for _ in range(2 * NBUF)]
            + [pltpu.SemaphoreType.DMA for _ in range(NSUB)]
        ),
    )
    def body(idx_hbm, table_hbm, out_hbm, *bufs):
        idxs = bufs[0:NBUF]
        rows = bufs[NBUF:2 * NBUF]
        tbufs = bufs[2 * NBUF:2 * NBUF + NSUB]
        isems = bufs[2 * NBUF + NSUB:3 * NBUF + NSUB]
        gsems = bufs[3 * NBUF + NSUB:4 * NBUF + NSUB]
        wsems = bufs[4 * NBUF + NSUB:4 * NBUF + 2 * NSUB]
        wid = lax.axis_index("s") * NC + lax.axis_index("c")
        t0 = wid * NSUB
        c0 = t0 * TB

        iot = lax.iota(jnp.int32, 16)
        gidx = [(fb * 16 + iot) // 8 for fb in range(D // 16)]
        ridx = [(fb * 16 + iot) % 8 for fb in range(D // 16)]

        for b in range(NBUF):
            pltpu.sync_copy(idx_hbm.at[b, pl.ds(c0, CT)], idxs[b])
            pltpu.async_copy(table_hbm.at[idxs[b]], rows[b], gsems[b])

        def step(grp, _):
            for b in range(NBUF):
                j = grp * NBUF + b
                pltpu.make_async_copy(
                    table_hbm.at[idxs[b]], rows[b], gsems[b]).wait()

                @pl.when(j + NBUF < S)
                def _():
                    pltpu.async_copy(
                        idx_hbm.at[j + NBUF, pl.ds(c0, CT)],
                        idxs[b], isems[b])

                for sub in range(NSUB):
                    @pl.when(j >= 1)
                    def _():
                        pltpu.make_async_copy(
                            tbufs[sub].at[:, :, pl.ds(0, TB)],
                            out_hbm.at[j, :, t0 + sub], wsems[sub]).wait()

                    def tloop(c, _c):
                        csp = iot * 0 + c
                        for fb in range(D // 16):
                            v = rows[b][sub * TB + c, pl.ds(fb * 16, 16)]
                            plsc.store_scatter(
                                tbufs[sub], [gidx[fb], ridx[fb], csp], v)
                        return _c

                    lax.fori_loop(0, TB, tloop, None, unroll=16)

                    pltpu.async_copy(
                        tbufs[sub].at[:, :, pl.ds(0, TB)],
                        out_hbm.at[j, :, t0 + sub], wsems[sub])

                @pl.when(j + NBUF < S)
                def _():
                    pltpu.make_async_copy(
                        idx_hbm.at[j + NBUF, pl.ds(c0, CT)],
                        idxs[b], isems[b]).wait()
                    pltpu.async_copy(
                        table_hbm.at[idxs[b]], rows[b], gsems[b])
            return _

        lax.fori_loop(0, S // NBUF, step, None)

        for sub in range(NSUB):
            pltpu.make_async_copy(
                tbufs[sub].at[:, :, pl.ds(0, TB)],
                out_hbm.at[S - 1, :, t0 + sub], wsems[sub]).wait()

    return body


def kernel(token_ids, embedding):
    T, S = token_ids.shape
    idx = token_ids.T.astype(jnp.int32)
    out5 = _gather_kernel(S, T)(idx, embedding)
    return jnp.reshape(jnp.transpose(out5, (2, 4, 0, 1, 3)), (T, S, D))


# speedup vs baseline: 2.6959x; 1.0112x over previous
import functools

import jax
import jax.numpy as jnp
from jax import lax
from jax.experimental import pallas as pl
from jax.experimental.pallas import tpu as pltpu
from jax.experimental.pallas import tpu_sc as plsc

NC = 2
NS = 16
NW = NC * NS
D = 64
TB = 128
NBUF = 4


def _gather_kernel(S, T):
    n_tb = T // TB
    B = S * T
    bw = B // NW
    n_ch = bw // TB
    assert bw % TB == 0 and n_ch % NBUF == 0
    mesh = plsc.VectorSubcoreMesh(
        core_axis_name="c", subcore_axis_name="s",
        num_cores=NC, num_subcores=NS)

    @functools.partial(
        pl.kernel,
        out_type=jax.ShapeDtypeStruct((S, D // 8, n_tb, 8, TB), jnp.float32),
        mesh=mesh,
        compiler_params=pltpu.CompilerParams(
            use_tc_tiling_on_sc=False, needs_layout_passes=False),
        scratch_types=(
            [pltpu.VMEM((n_ch * TB,), jnp.int32)]
            + [pltpu.VMEM((TB, D), jnp.float32) for _ in range(NBUF)]
            + [pltpu.VMEM((D, TB + 1), jnp.float32) for _ in range(NBUF)]
            + [pltpu.SemaphoreType.DMA for _ in range(2 * NBUF)]
        ),
    )
    def body(idx_hbm, table_hbm, out_hbm, idx_all, *bufs):
        rows = bufs[0:NBUF]
        tbufs = bufs[NBUF:2 * NBUF]
        gsems = bufs[2 * NBUF:3 * NBUF]
        wsems = bufs[3 * NBUF:4 * NBUF]
        wid = lax.axis_index("s") * NC + lax.axis_index("c")
        base_q = wid * n_ch

        iot = lax.iota(jnp.int32, 16)
        fidx = [fb * 16 + iot for fb in range(D // 16)]

        pltpu.sync_copy(idx_hbm.at[pl.ds(base_q * TB, n_ch * TB)], idx_all)

        for b in range(NBUF):
            pltpu.async_copy(
                table_hbm.at[idx_all.at[pl.ds(b * TB, TB)]],
                rows[b], gsems[b])

        def step(grp, _):
            for b in range(NBUF):
                i = grp * NBUF + b
                q = base_q + i
                j = q // n_tb
                t = q % n_tb
                pltpu.make_async_copy(
                    table_hbm.at[idx_all.at[pl.ds(i * TB, TB)]],
                    rows[b], gsems[b]).wait()

                @pl.when(i >= NBUF)
                def _():
                    for g in range(D // 8):
                        pltpu.make_async_copy(
                            tbufs[b].at[pl.ds(g * 8, 8), pl.ds(0, TB)],
                            out_hbm.at[j, g, t], wsems[b]).wait()

                def tloop(c, _c):
                    csp = iot * 0 + c
                    for fb in range(D // 16):
                        v = rows[b][c, pl.ds(fb * 16, 16)]
                        plsc.store_scatter(tbufs[b], [fidx[fb], csp], v)
                    return _c

                lax.fori_loop(0, TB, tloop, None, unroll=8)

                for g in range(D // 8):
                    pltpu.async_copy(
                        tbufs[b].at[pl.ds(g * 8, 8), pl.ds(0, TB)],
                        out_hbm.at[j, g, t], wsems[b])

                @pl.when(i + NBUF < n_ch)
                def _():
                    pltpu.async_copy(
                        table_hbm.at[idx_all.at[pl.ds((i + NBUF) * TB, TB)]],
                        rows[b], gsems[b])
            return _

        lax.fori_loop(0, n_ch // NBUF, step, None)

        for b in range(NBUF):
            q = base_q + n_ch - NBUF + b
            j = q // n_tb
            t = q % n_tb
            for g in range(D // 8):
                pltpu.make_async_copy(
                    tbufs[b].at[pl.ds(g * 8, 8), pl.ds(0, TB)],
                    out_hbm.at[j, g, t], wsems[b]).wait()

    return body


def kernel(token_ids, embedding):
    T, S = token_ids.shape
    B = T * S
    idx = jnp.reshape(token_ids.T, (B,)).astype(jnp.int32)
    out5 = _gather_kernel(S, T)(idx, embedding)
    return jnp.reshape(jnp.transpose(out5, (2, 4, 0, 1, 3)), (T, S, D))
